# baseline (speedup 1.0000x reference)
import functools

import jax
import jax.numpy as jnp
import numpy as np
from jax import lax
from jax.experimental import pallas as pl
from jax.experimental.pallas import tpu as pltpu
from jax.experimental.pallas import tpu_sc as plsc

_ALPHA = 4
_NC, _NS = 2, 16
_NW = _NC * _NS


def _tc_fast_copy(frames):
    C, T, H, W = frames.shape
    n = T // _ALPHA

    def _body(in_hbm, fast_hbm, bufs, sem_in, sem_fast):
        def in_dma(j):
            return pltpu.make_async_copy(
                in_hbm.at[:, pl.ds(j * _ALPHA, _ALPHA)],
                bufs.at[j],
                sem_in.at[j],
            )

        def fast_dma(j):
            return pltpu.make_async_copy(
                bufs.at[j],
                fast_hbm.at[:, pl.ds(j * _ALPHA, _ALPHA)],
                sem_fast.at[j],
            )

        for j in range(n):
            in_dma(j).start()
        for j in range(n):
            in_dma(j).wait()
            fast_dma(j).start()
        for j in range(n):
            fast_dma(j).wait()

    return pl.pallas_call(
        _body,
        in_specs=[pl.BlockSpec(memory_space=pltpu.MemorySpace.HBM)],
        out_specs=pl.BlockSpec(memory_space=pltpu.MemorySpace.HBM),
        out_shape=jax.ShapeDtypeStruct((C, T, H, W), frames.dtype),
        scratch_shapes=[
            pltpu.VMEM((n, C, _ALPHA, H, W), frames.dtype),
            pltpu.SemaphoreType.DMA((n,)),
            pltpu.SemaphoreType.DMA((n,)),
        ],
    )(frames)


def _sc_gather(frames2d, C, T, n, cols):
    rows_out = C * n
    nchunk = max(1, _NW // rows_out * 2)
    while (rows_out * nchunk) % _NW or cols % nchunk or (cols // nchunk) % 8:
        nchunk += 1
    cw = cols // nchunk
    tpw = rows_out * nchunk // _NW

    mesh = plsc.VectorSubcoreMesh(core_axis_name="c", subcore_axis_name="s")

    @functools.partial(
        pl.kernel,
        mesh=mesh,
        out_type=jax.ShapeDtypeStruct((rows_out, cols), frames2d.dtype),
    )
    def k(in_hbm, out_hbm):
        wid = lax.axis_index("s") * _NC + lax.axis_index("c")
        for i in range(tpw):
            t = wid * tpw + i
            row = t // nchunk
            chunk = t % nchunk
            j = row % n
            c = row // n
            src = c * T + (j * (T - 1)) // (n - 1)
            pltpu.sync_copy(
                in_hbm.at[src, pl.ds(chunk * cw, cw)],
                out_hbm.at[row, pl.ds(chunk * cw, cw)],
            )

    return k(frames2d)


def kernel(frames):
    C, T, H, W = frames.shape
    n = T // _ALPHA
    idx = np.linspace(0.0, T - 1, n).astype(np.int32)
    assert all(int(t) == (j * (T - 1)) // (n - 1) for j, t in enumerate(idx))

    fast = _tc_fast_copy(frames)
    slow2d = _sc_gather(frames.reshape(C * T, H * W), C, T, n, H * W)
    return (slow2d.reshape(C, n, H, W), fast)


# speedup vs baseline: 3.6927x; 3.6927x over previous
import functools

import jax
import jax.numpy as jnp
import numpy as np
from jax import lax
from jax.experimental import pallas as pl
from jax.experimental.pallas import tpu as pltpu
from jax.experimental.pallas import tpu_sc as plsc

_ALPHA = 4
_NC, _NS = 2, 16
_NW = _NC * _NS


def _tc_fast_copy(frames):
    C, T, H, W = frames.shape
    n = T // _ALPHA

    def _body(in_hbm, fast_hbm, bufs, sem_in, sem_fast):
        def in_dma(j):
            return pltpu.make_async_copy(
                in_hbm.at[:, pl.ds(j * _ALPHA, _ALPHA)],
                bufs.at[j],
                sem_in.at[j],
            )

        def fast_dma(j):
            return pltpu.make_async_copy(
                bufs.at[j],
                fast_hbm.at[:, pl.ds(j * _ALPHA, _ALPHA)],
                sem_fast.at[j],
            )

        for j in range(n):
            in_dma(j).start()
        for j in range(n):
            in_dma(j).wait()
            fast_dma(j).start()
        for j in range(n):
            fast_dma(j).wait()

    return pl.pallas_call(
        _body,
        in_specs=[pl.BlockSpec(memory_space=pltpu.MemorySpace.HBM)],
        out_specs=pl.BlockSpec(memory_space=pltpu.MemorySpace.HBM),
        out_shape=jax.ShapeDtypeStruct((C, T, H, W), frames.dtype),
        scratch_shapes=[
            pltpu.VMEM((n, C, _ALPHA, H, W), frames.dtype),
            pltpu.SemaphoreType.DMA((n,)),
            pltpu.SemaphoreType.DMA((n,)),
        ],
    )(frames)


def _sc_gather(frames2d, C, T, n, cols):
    rows_out = C * n
    nchunk = max(1, _NW // rows_out * 2)
    while (rows_out * nchunk) % _NW or cols % nchunk or (cols // nchunk) % 8:
        nchunk += 1
    cw = cols // nchunk
    tpw = rows_out * nchunk // _NW

    mesh = plsc.VectorSubcoreMesh(core_axis_name="c", subcore_axis_name="s")

    @functools.partial(
        pl.kernel,
        mesh=mesh,
        out_type=jax.ShapeDtypeStruct((rows_out, cols), frames2d.dtype),
        scratch_types=[
            pltpu.VMEM((2, cw), frames2d.dtype),
            pltpu.SemaphoreType.DMA((2,)),
            pltpu.SemaphoreType.DMA((2,)),
        ],
    )
    def k(in_hbm, out_hbm, bufs, sem_in, sem_out):
        wid = lax.axis_index("s") * _NC + lax.axis_index("c")

        def task(i):
            t = wid * tpw + i
            row = t // nchunk
            chunk = t % nchunk
            j = row % n
            c = row // n
            src = c * T + (j * (T - 1)) // (n - 1)
            col = chunk * cw
            return src, row, col

        def in_dma(i):
            src, _, col = task(i)
            return pltpu.make_async_copy(
                in_hbm.at[src, pl.ds(col, cw)], bufs.at[i % 2], sem_in.at[i % 2]
            )

        def out_dma(i):
            _, row, col = task(i)
            return pltpu.make_async_copy(
                bufs.at[i % 2], out_hbm.at[row, pl.ds(col, cw)], sem_out.at[i % 2]
            )

        in_dma(0).start()
        if tpw > 1:
            in_dma(1).start()
        for i in range(tpw):
            in_dma(i).wait()
            out_dma(i).start()
            if i + 2 < tpw:
                out_dma(i).wait()
                in_dma(i + 2).start()
        for i in range(max(0, tpw - 2), tpw):
            out_dma(i).wait()

    return k(frames2d)


def kernel(frames):
    C, T, H, W = frames.shape
    n = T // _ALPHA
    idx = np.linspace(0.0, T - 1, n).astype(np.int32)
    assert all(int(t) == (j * (T - 1)) // (n - 1) for j, t in enumerate(idx))

    fast = _tc_fast_copy(frames)
    slow2d = _sc_gather(frames.reshape(C * T, H * W), C, T, n, H * W)
    return (slow2d.reshape(C, n, H, W), fast)


# speedup vs baseline: 6.9702x; 1.8876x over previous
import functools

import jax
import jax.numpy as jnp
import numpy as np
from jax import lax
from jax.experimental import pallas as pl
from jax.experimental.pallas import tpu as pltpu
from jax.experimental.pallas import tpu_sc as plsc

_ALPHA = 4
_NC, _NS = 2, 16
_NW = _NC * _NS


def _tc_fast_copy(frames):
    C, T, H, W = frames.shape
    n = T // _ALPHA

    def _body(in_hbm, fast_hbm, bufs, sem_in, sem_fast):
        def in_dma(j):
            return pltpu.make_async_copy(
                in_hbm.at[:, pl.ds(j * _ALPHA, _ALPHA)],
                bufs.at[j],
                sem_in.at[j],
            )

        def fast_dma(j):
            return pltpu.make_async_copy(
                bufs.at[j],
                fast_hbm.at[:, pl.ds(j * _ALPHA, _ALPHA)],
                sem_fast.at[j],
            )

        for j in range(n):
            in_dma(j).start()
        for j in range(n):
            in_dma(j).wait()
            fast_dma(j).start()
        for j in range(n):
            fast_dma(j).wait()

    return pl.pallas_call(
        _body,
        in_specs=[pl.BlockSpec(memory_space=pltpu.MemorySpace.HBM)],
        out_specs=pl.BlockSpec(memory_space=pltpu.MemorySpace.HBM),
        out_shape=jax.ShapeDtypeStruct((C, T, H, W), frames.dtype),
        scratch_shapes=[
            pltpu.VMEM((n, C, _ALPHA, H, W), frames.dtype),
            pltpu.SemaphoreType.DMA((n,)),
            pltpu.SemaphoreType.DMA((n,)),
        ],
    )(frames)


def _sc_gather(frames, C, T, n):
    _, _, H, W = frames.shape
    planes = C * n
    nchunk = 1
    while (planes * nchunk) % _NW or H % nchunk or (H // nchunk) % 8:
        nchunk += 1
    hh = H // nchunk
    tpw = planes * nchunk // _NW

    mesh = plsc.VectorSubcoreMesh(core_axis_name="c", subcore_axis_name="s")

    @functools.partial(
        pl.kernel,
        mesh=mesh,
        out_type=jax.ShapeDtypeStruct((C, n, H, W), frames.dtype),
        scratch_types=[
            pltpu.VMEM((2, hh, W), frames.dtype),
            pltpu.SemaphoreType.DMA((2,)),
            pltpu.SemaphoreType.DMA((2,)),
        ],
        compiler_params=pltpu.CompilerParams(use_tc_tiling_on_sc=True),
    )
    def k(in_hbm, out_hbm, bufs, sem_in, sem_out):
        wid = lax.axis_index("s") * _NC + lax.axis_index("c")

        def task(i):
            t = wid * tpw + i
            plane = t // nchunk
            chunk = t % nchunk
            j = plane % n
            c = plane // n
            src = (j * (T - 1)) // (n - 1)
            return c, src, j, chunk * hh

        def in_dma(i):
            c, src, _, h0 = task(i)
            return pltpu.make_async_copy(
                in_hbm.at[c, src, pl.ds(h0, hh)], bufs.at[i % 2], sem_in.at[i % 2]
            )

        def out_dma(i):
            c, _, j, h0 = task(i)
            return pltpu.make_async_copy(
                bufs.at[i % 2], out_hbm.at[c, j, pl.ds(h0, hh)], sem_out.at[i % 2]
            )

        in_dma(0).start()
        if tpw > 1:
            in_dma(1).start()
        for i in range(tpw):
            in_dma(i).wait()
            out_dma(i).start()
            if i + 2 < tpw:
                out_dma(i).wait()
                in_dma(i + 2).start()
        for i in range(max(0, tpw - 2), tpw):
            out_dma(i).wait()

    return k(frames)


def kernel(frames):
    C, T, H, W = frames.shape
    n = T // _ALPHA
    idx = np.linspace(0.0, T - 1, n).astype(np.int32)
    assert all(int(t) == (j * (T - 1)) // (n - 1) for j, t in enumerate(idx))

    fast = _tc_fast_copy(frames)
    slow = _sc_gather(frames, C, T, n)
    return (slow, fast)
